# baseline (speedup 1.0000x reference)
import jax
import jax.numpy as jnp
from jax import lax
from jax.experimental import pallas as pl
from jax.experimental.pallas import tpu as pltpu
from jax.experimental.pallas import tpu_sc as plsc

N_CORES = 2
N_SUBCORES = 16
NW = N_CORES * N_SUBCORES
CHUNK = 100
NBUF = 4


def _body(x_hbm, tgt_hbm, pos_hbm, out_hbm,
          idx0, idx1, idx2, idx3,
          rows0, rows1, rows2, rows3,
          psem0, psem1, psem2, psem3,
          isem0, isem1, isem2, isem3,
          gsem0, gsem1, gsem2, gsem3,
          osem0, osem1, osem2, osem3):
    cid = lax.axis_index("c")
    sid = lax.axis_index("s")
    wid = sid * N_CORES + cid
    nper = out_hbm.shape[0] // NW
    base = wid * nper

    idx = (idx0, idx1, idx2, idx3)
    rows = (rows0, rows1, rows2, rows3)
    psem = (psem0, psem1, psem2, psem3)
    isem = (isem0, isem1, isem2, isem3)
    gsem = (gsem0, gsem1, gsem2, gsem3)
    osem = (osem0, osem1, osem2, osem3)

    def issue_in(b, i):
        pltpu.async_copy(pos_hbm, rows[b], psem[b])
        pltpu.async_copy(x_hbm.at[i], idx[b], isem[b])

    def fire(b, i):
        pltpu.make_async_copy(pos_hbm, rows[b], psem[b]).wait()
        pltpu.make_async_copy(x_hbm.at[i], idx[b], isem[b]).wait()
        pltpu.async_copy(tgt_hbm.at[idx[b].at[0]], rows[b].at[pl.ds(0, CHUNK)],
                         gsem[b], add=True)
        pltpu.async_copy(tgt_hbm.at[idx[b].at[1]],
                         rows[b].at[pl.ds(CHUNK, CHUNK)], gsem[b], add=True)

    def drain(b, i):
        pltpu.make_async_copy(tgt_hbm.at[idx[b].at[0]],
                              rows[b].at[pl.ds(0, CHUNK)], gsem[b]).wait()
        pltpu.make_async_copy(tgt_hbm.at[idx[b].at[1]],
                              rows[b].at[pl.ds(CHUNK, CHUNK)], gsem[b]).wait()
        pltpu.async_copy(rows[b], out_hbm.at[i], osem[b])

    def wait_out(b, i):
        pltpu.make_async_copy(rows[b], out_hbm.at[i], osem[b]).wait()

    def round_fn(q, carry):
        r0 = base + NBUF * q
        for k in range(NBUF):
            @pl.when(q >= 1)
            def _(k=k, r0=r0):
                wait_out(k, r0 + k - NBUF)

            issue_in(k, r0 + k)

            if k >= 1:
                fire(k - 1, r0 + k - 1)
            else:
                @pl.when(q >= 1)
                def _(r0=r0):
                    fire(NBUF - 1, r0 - 1)
            if k >= 2:
                drain(k - 2, r0 + k - 2)
            else:
                @pl.when(q >= 1)
                def _(k=k, r0=r0):
                    drain((k - 2) % NBUF, r0 + k - 2)
        return carry

    lax.fori_loop(0, nper // NBUF, round_fn, 0)

    last = base + nper - 1
    fire(NBUF - 1, last)
    drain(NBUF - 2, last - 1)
    drain(NBUF - 1, last)
    for k in range(NBUF):
        wait_out(k, last - (NBUF - 1) + k)


def kernel(x, W_tgt, W_pos):
    B, S = x.shape
    D = W_tgt.shape[1]
    x3 = x.reshape(B, S // CHUNK, CHUNK).astype(jnp.int32)
    mesh = plsc.VectorSubcoreMesh(core_axis_name="c", subcore_axis_name="s")
    f = pl.kernel(
        _body,
        mesh=mesh,
        compiler_params=pltpu.CompilerParams(use_tc_tiling_on_sc=False),
        out_type=jax.ShapeDtypeStruct((B, S, D), jnp.float32),
        scratch_types=(
            [pltpu.VMEM((S // CHUNK, CHUNK), jnp.int32)] * NBUF
            + [pltpu.VMEM((S, D), jnp.float32)] * NBUF
            + [pltpu.SemaphoreType.DMA] * (4 * NBUF)
        ),
    )
    return f(x3, W_tgt, W_pos)


# speedup vs baseline: 1.2570x; 1.2570x over previous
import jax
import jax.numpy as jnp
from jax import lax
from jax.experimental import pallas as pl
from jax.experimental.pallas import tpu as pltpu
from jax.experimental.pallas import tpu_sc as plsc

N_CORES = 2
N_SUBCORES = 16
NW = N_CORES * N_SUBCORES
CHUNK = 100
NBUF = 4


def _body(x_hbm, tgt_hbm, pos_hbm, out_hbm, pos_sh,
          idx0, idx1, idx2, idx3,
          rows0, rows1, rows2, rows3,
          isem0, isem1, isem2, isem3,
          gsem0, gsem1, gsem2, gsem3,
          osem0, osem1, osem2, osem3):
    cid = lax.axis_index("c")
    sid = lax.axis_index("s")
    wid = sid * N_CORES + cid
    nper = out_hbm.shape[0] // NW
    base = wid * nper

    idx = (idx0, idx1, idx2, idx3)
    rows = (rows0, rows1, rows2, rows3)
    isem = (isem0, isem1, isem2, isem3)
    gsem = (gsem0, gsem1, gsem2, gsem3)
    osem = (osem0, osem1, osem2, osem3)

    @pl.when(sid == 0)
    def _():
        pltpu.sync_copy(pos_hbm, pos_sh)

    plsc.subcore_barrier()

    def issue_in(b, i):
        pltpu.sync_copy(pos_sh, rows[b])
        pltpu.async_copy(x_hbm.at[i], idx[b], isem[b])

    def fire(b, i):
        pltpu.make_async_copy(x_hbm.at[i], idx[b], isem[b]).wait()
        pltpu.async_copy(tgt_hbm.at[idx[b].at[0]], rows[b].at[pl.ds(0, CHUNK)],
                         gsem[b], add=True)
        pltpu.async_copy(tgt_hbm.at[idx[b].at[1]],
                         rows[b].at[pl.ds(CHUNK, CHUNK)], gsem[b], add=True)

    def drain(b, i):
        pltpu.make_async_copy(tgt_hbm.at[idx[b].at[0]],
                              rows[b].at[pl.ds(0, CHUNK)], gsem[b]).wait()
        pltpu.make_async_copy(tgt_hbm.at[idx[b].at[1]],
                              rows[b].at[pl.ds(CHUNK, CHUNK)], gsem[b]).wait()
        pltpu.async_copy(rows[b], out_hbm.at[i], osem[b])

    def wait_out(b, i):
        pltpu.make_async_copy(rows[b], out_hbm.at[i], osem[b]).wait()

    def round_fn(q, carry):
        r0 = base + NBUF * q
        for k in range(NBUF):
            @pl.when(q >= 1)
            def _(k=k, r0=r0):
                wait_out(k, r0 + k - NBUF)

            issue_in(k, r0 + k)

            if k >= 1:
                fire(k - 1, r0 + k - 1)
            else:
                @pl.when(q >= 1)
                def _(r0=r0):
                    fire(NBUF - 1, r0 - 1)
            if k >= 2:
                drain(k - 2, r0 + k - 2)
            else:
                @pl.when(q >= 1)
                def _(k=k, r0=r0):
                    drain((k - 2) % NBUF, r0 + k - 2)
        return carry

    lax.fori_loop(0, nper // NBUF, round_fn, 0)

    last = base + nper - 1
    fire(NBUF - 1, last)
    drain(NBUF - 2, last - 1)
    drain(NBUF - 1, last)
    for k in range(NBUF):
        wait_out(k, last - (NBUF - 1) + k)


def kernel(x, W_tgt, W_pos):
    B, S = x.shape
    D = W_tgt.shape[1]
    x3 = x.reshape(B, S // CHUNK, CHUNK).astype(jnp.int32)
    mesh = plsc.VectorSubcoreMesh(core_axis_name="c", subcore_axis_name="s")
    f = pl.kernel(
        _body,
        mesh=mesh,
        compiler_params=pltpu.CompilerParams(use_tc_tiling_on_sc=False),
        out_type=jax.ShapeDtypeStruct((B, S, D), jnp.float32),
        scratch_types=(
            [pltpu.VMEM_SHARED((S, D), jnp.float32)]
            + [pltpu.VMEM((S // CHUNK, CHUNK), jnp.int32)] * NBUF
            + [pltpu.VMEM((S, D), jnp.float32)] * NBUF
            + [pltpu.SemaphoreType.DMA] * (3 * NBUF)
        ),
    )
    return f(x3, W_tgt, W_pos)


# speedup vs baseline: 1.2693x; 1.0099x over previous
import jax
import jax.numpy as jnp
from jax import lax
from jax.experimental import pallas as pl
from jax.experimental.pallas import tpu as pltpu
from jax.experimental.pallas import tpu_sc as plsc

N_CORES = 2
N_SUBCORES = 16
NW = N_CORES * N_SUBCORES
CHUNK = 100
G = 4
NBUF = 4


def _body(x_hbm, tgt_hbm, pos_hbm, out_hbm, pos_sh,
          idx0, idx1, idx2, idx3,
          rows0, rows1, rows2, rows3,
          isem0, isem1, isem2, isem3,
          gsem0, gsem1, gsem2, gsem3,
          osem0, osem1, osem2, osem3):
    cid = lax.axis_index("c")
    sid = lax.axis_index("s")
    wid = sid * N_CORES + cid
    nchunks = out_hbm.shape[0]
    nper = nchunks // NW
    base = wid * nper

    idx = (idx0, idx1, idx2, idx3)
    rows = (rows0, rows1, rows2, rows3)
    isem = (isem0, isem1, isem2, isem3)
    gsem = (gsem0, gsem1, gsem2, gsem3)
    osem = (osem0, osem1, osem2, osem3)

    @pl.when(sid == 0)
    def _():
        for r in range(G // 2):
            pltpu.sync_copy(pos_hbm, pos_sh.at[pl.ds(2 * r, 2)])

    plsc.subcore_barrier()

    def issue_in(b, j):
        pltpu.sync_copy(pos_sh, rows[b])
        pltpu.async_copy(x_hbm.at[pl.ds(j, G)], idx[b], isem[b])

    def fire(b, j):
        pltpu.make_async_copy(x_hbm.at[pl.ds(j, G)], idx[b], isem[b]).wait()
        for t in range(G):
            pltpu.async_copy(tgt_hbm.at[idx[b].at[t]], rows[b].at[t],
                             gsem[b], add=True)

    def drain(b, j):
        for t in range(G):
            pltpu.make_async_copy(tgt_hbm.at[idx[b].at[t]], rows[b].at[t],
                                  gsem[b]).wait()
        pltpu.async_copy(rows[b], out_hbm.at[pl.ds(j, G)], osem[b])

    def wait_out(b, j):
        pltpu.make_async_copy(rows[b], out_hbm.at[pl.ds(j, G)], osem[b]).wait()

    def round_fn(q, carry):
        j0 = base + NBUF * G * q
        for k in range(NBUF):
            jk = j0 + G * k

            @pl.when(q >= 1)
            def _(k=k, jk=jk):
                wait_out(k, jk - NBUF * G)

            issue_in(k, jk)

            if k >= 1:
                fire(k - 1, jk - G)
            else:
                @pl.when(q >= 1)
                def _(jk=jk):
                    fire(NBUF - 1, jk - G)
            if k >= 2:
                drain(k - 2, jk - 2 * G)
            else:
                @pl.when(q >= 1)
                def _(k=k, jk=jk):
                    drain((k - 2) % NBUF, jk - 2 * G)
        return carry

    lax.fori_loop(0, nper // (NBUF * G), round_fn, 0)

    end = base + nper
    fire(NBUF - 1, end - G)
    drain(NBUF - 2, end - 2 * G)
    drain(NBUF - 1, end - G)
    for k in range(NBUF):
        wait_out(k, end - (NBUF - k) * G)


def kernel(x, W_tgt, W_pos):
    B, S = x.shape
    D = W_tgt.shape[1]
    nch = S // CHUNK
    x2 = x.reshape(B * nch, CHUNK).astype(jnp.int32)
    pos2 = W_pos.reshape(nch, CHUNK, D)
    mesh = plsc.VectorSubcoreMesh(core_axis_name="c", subcore_axis_name="s")
    f = pl.kernel(
        _body,
        mesh=mesh,
        compiler_params=pltpu.CompilerParams(use_tc_tiling_on_sc=False),
        out_type=jax.ShapeDtypeStruct((B * nch, CHUNK, D), jnp.float32),
        scratch_types=(
            [pltpu.VMEM_SHARED((G, CHUNK, D), jnp.float32)]
            + [pltpu.VMEM((G, CHUNK), jnp.int32)] * NBUF
            + [pltpu.VMEM((G, CHUNK, D), jnp.float32)] * NBUF
            + [pltpu.SemaphoreType.DMA] * (3 * NBUF)
        ),
    )
    return f(x2, W_tgt, pos2).reshape(B, S, D)
